# Initial kernel scaffold; baseline (speedup 1.0000x reference)
import jax
import jax.numpy as jnp
from jax.experimental import pallas as pl


def kernel(inputs, emb_table, W, b):
    raise NotImplementedError("write your pallas kernel here")


# speedup vs baseline: 1.5193x; 1.5193x over previous
import functools

import jax
import jax.numpy as jnp
from jax import lax
from jax.experimental import pallas as pl
from jax.experimental.pallas import tpu as pltpu
from jax.experimental.pallas import tpu_sc as plsc

VOCAB = 100000
D = 9
DP = 16
B = 1024
H = 200
VB = 2048
NVB = (VOCAB + VB - 1) // VB

NC = 2
NS = 16
NW = NC * NS
BPW = B // NW
IPW = BPW * H


def _sc_gather_mean(inputs_flat, tab16):
    mesh = plsc.VectorSubcoreMesh(core_axis_name="c", subcore_axis_name="s")

    @functools.partial(
        pl.kernel,
        mesh=mesh,
        out_type=jax.ShapeDtypeStruct((B, DP), jnp.float32),
        compiler_params=pltpu.CompilerParams(use_tc_tiling_on_sc=False),
        scratch_types=[
            pltpu.VMEM((IPW,), jnp.int32),
            pltpu.VMEM((IPW, DP), jnp.float32),
            pltpu.VMEM((BPW, DP), jnp.float32),
            pltpu.SemaphoreType.DMA,
        ],
    )
    def gm(idx_hbm, tab_hbm, out_hbm, idx_v, rows_v, e_v, sem):
        wid = lax.axis_index("s") * NC + lax.axis_index("c")
        pltpu.sync_copy(idx_hbm.at[pl.ds(wid * IPW, IPW)], idx_v)
        pltpu.async_copy(tab_hbm.at[idx_v], rows_v, sem).wait()

        def batch_body(i, carry):
            def inner(r, acc):
                return acc + rows_v[i * H + r]

            acc = lax.fori_loop(0, H, inner, jnp.zeros((DP,), jnp.float32))
            e_v[i] = acc * (1.0 / H)
            return carry

        lax.fori_loop(0, BPW, batch_body, 0)
        pltpu.sync_copy(e_v, out_hbm.at[pl.ds(wid * BPW, BPW)])

    return gm(inputs_flat, tab16)


def _p1_body(e_ref, w_ref, b_ref, m_ref, s_ref):
    j = pl.program_id(0)

    @pl.when(j == 0)
    def _():
        m_ref[...] = jnp.full_like(m_ref, -jnp.inf)
        s_ref[...] = jnp.zeros_like(s_ref)

    logits = lax.dot_general(
        e_ref[...], w_ref[...], (((1,), (1,)), ((), ())),
        preferred_element_type=jnp.float32) + b_ref[...]
    col = j * VB + lax.broadcasted_iota(jnp.int32, (1, VB), 1)
    logits = jnp.where(col < VOCAB, logits, -jnp.inf)
    bm = jnp.max(logits, axis=1, keepdims=True)
    m_old = m_ref[...]
    m_new = jnp.maximum(m_old, bm)
    s_ref[...] = s_ref[...] * jnp.exp(m_old - m_new) + jnp.sum(
        jnp.exp(logits - m_new), axis=1, keepdims=True)
    m_ref[...] = m_new


def _p2_body(e_ref, w_ref, b_ref, m_ref, s_ref, o_ref):
    logits = lax.dot_general(
        e_ref[...], w_ref[...], (((1,), (1,)), ((), ())),
        preferred_element_type=jnp.float32) + b_ref[...]
    o_ref[...] = logits - (m_ref[...] + jnp.log(s_ref[...]))


def _tc_pass1(e, w16, b2):
    return pl.pallas_call(
        _p1_body,
        grid=(NVB,),
        in_specs=[
            pl.BlockSpec((B, DP), lambda j: (0, 0)),
            pl.BlockSpec((VB, DP), lambda j: (j, 0)),
            pl.BlockSpec((1, VB), lambda j: (0, j)),
        ],
        out_specs=[
            pl.BlockSpec((B, 1), lambda j: (0, 0)),
            pl.BlockSpec((B, 1), lambda j: (0, 0)),
        ],
        out_shape=[
            jax.ShapeDtypeStruct((B, 1), jnp.float32),
            jax.ShapeDtypeStruct((B, 1), jnp.float32),
        ],
    )(e, w16, b2)


def _tc_pass2(e, w16, b2, m, s):
    return pl.pallas_call(
        _p2_body,
        grid=(NVB,),
        in_specs=[
            pl.BlockSpec((B, DP), lambda j: (0, 0)),
            pl.BlockSpec((VB, DP), lambda j: (j, 0)),
            pl.BlockSpec((1, VB), lambda j: (0, j)),
            pl.BlockSpec((B, 1), lambda j: (0, 0)),
            pl.BlockSpec((B, 1), lambda j: (0, 0)),
        ],
        out_specs=pl.BlockSpec((B, VB), lambda j: (0, j)),
        out_shape=jax.ShapeDtypeStruct((B, VOCAB), jnp.float32),
    )(e, w16, b2, m, s)


def kernel(inputs, emb_table, W, b):
    tab16 = jnp.pad(emb_table, ((0, 0), (0, DP - D)))
    w16 = jnp.pad(W, ((0, 0), (0, DP - D)))
    b2 = b.reshape(1, VOCAB)
    idx = inputs.astype(jnp.int32).reshape(-1)
    e = _sc_gather_mean(idx, tab16)
    m, s = _tc_pass1(e, w16, b2)
    return _tc_pass2(e, w16, b2, m, s)


# speedup vs baseline: 1.5481x; 1.0190x over previous
import functools

import jax
import jax.numpy as jnp
from jax import lax
from jax.experimental import pallas as pl
from jax.experimental.pallas import tpu as pltpu
from jax.experimental.pallas import tpu_sc as plsc

VOCAB = 100000
D = 9
DP = 16
B = 1024
H = 200
VB = 4096
NVB = (VOCAB + VB - 1) // VB
VP = NVB * VB

NC = 2
NS = 16
NW = NC * NS
BPW = B // NW
IPW = BPW * H


def _sc_gather_mean(inputs_flat, tab16):
    mesh = plsc.VectorSubcoreMesh(core_axis_name="c", subcore_axis_name="s")

    @functools.partial(
        pl.kernel,
        mesh=mesh,
        out_type=jax.ShapeDtypeStruct((B, DP), jnp.float32),
        compiler_params=pltpu.CompilerParams(use_tc_tiling_on_sc=False),
        scratch_types=[
            pltpu.VMEM((IPW,), jnp.int32),
            pltpu.VMEM((IPW, DP), jnp.float32),
            pltpu.VMEM((BPW, DP), jnp.float32),
            pltpu.SemaphoreType.DMA,
        ],
    )
    def gm(idx_hbm, tab_hbm, out_hbm, idx_v, rows_v, e_v, sem):
        wid = lax.axis_index("s") * NC + lax.axis_index("c")
        pltpu.sync_copy(idx_hbm.at[pl.ds(wid * IPW, IPW)], idx_v)
        pltpu.async_copy(tab_hbm.at[idx_v], rows_v, sem).wait()

        def batch_body(i, carry):
            def inner(r, acc):
                return acc + rows_v[i * H + r]

            acc = lax.fori_loop(0, H, inner, jnp.zeros((DP,), jnp.float32))
            e_v[i] = acc * (1.0 / H)
            return carry

        lax.fori_loop(0, BPW, batch_body, 0)
        pltpu.sync_copy(e_v, out_hbm.at[pl.ds(wid * BPW, BPW)])

    return gm(inputs_flat, tab16)


def _p1_body(e_ref, w_ref, b_ref, m_ref, s_ref):
    j = pl.program_id(0)

    @pl.when(j == 0)
    def _():
        m_ref[...] = jnp.full_like(m_ref, -jnp.inf)
        s_ref[...] = jnp.zeros_like(s_ref)

    logits = lax.dot_general(
        e_ref[...], w_ref[...], (((1,), (1,)), ((), ())),
        preferred_element_type=jnp.float32) + b_ref[...]
    bm = jnp.max(logits, axis=1, keepdims=True)
    m_old = m_ref[...]
    m_new = jnp.maximum(m_old, bm)
    s_ref[...] = s_ref[...] * jnp.exp(m_old - m_new) + jnp.sum(
        jnp.exp(logits - m_new), axis=1, keepdims=True)
    m_ref[...] = m_new


def _p2_body(e_ref, w_ref, b_ref, m_ref, s_ref, o_ref):
    logits = lax.dot_general(
        e_ref[...], w_ref[...], (((1,), (1,)), ((), ())),
        preferred_element_type=jnp.float32) + b_ref[...]
    o_ref[...] = logits - (m_ref[...] + jnp.log(s_ref[...]))


def _tc_pass1(e, w16, b2):
    return pl.pallas_call(
        _p1_body,
        grid=(NVB,),
        in_specs=[
            pl.BlockSpec((B, DP), lambda j: (0, 0)),
            pl.BlockSpec((VB, DP), lambda j: (j, 0)),
            pl.BlockSpec((1, VB), lambda j: (0, j)),
        ],
        out_specs=[
            pl.BlockSpec((B, 1), lambda j: (0, 0)),
            pl.BlockSpec((B, 1), lambda j: (0, 0)),
        ],
        out_shape=[
            jax.ShapeDtypeStruct((B, 1), jnp.float32),
            jax.ShapeDtypeStruct((B, 1), jnp.float32),
        ],
    )(e, w16, b2)


def _tc_pass2(e, w16, b2, m, s):
    return pl.pallas_call(
        _p2_body,
        grid=(NVB,),
        in_specs=[
            pl.BlockSpec((B, DP), lambda j: (0, 0)),
            pl.BlockSpec((VB, DP), lambda j: (j, 0)),
            pl.BlockSpec((1, VB), lambda j: (0, j)),
            pl.BlockSpec((B, 1), lambda j: (0, 0)),
            pl.BlockSpec((B, 1), lambda j: (0, 0)),
        ],
        out_specs=pl.BlockSpec((B, VB), lambda j: (0, j)),
        out_shape=jax.ShapeDtypeStruct((B, VOCAB), jnp.float32),
    )(e, w16, b2, m, s)


def kernel(inputs, emb_table, W, b):
    tab16 = jnp.pad(emb_table, ((0, 0), (0, DP - D)))
    w16 = jnp.pad(W, ((0, VP - VOCAB), (0, DP - D)))
    b2 = jnp.pad(b, (0, VP - VOCAB), constant_values=-jnp.inf).reshape(1, VP)
    idx = inputs.astype(jnp.int32).reshape(-1)
    e = _sc_gather_mean(idx, tab16)
    m, s = _tc_pass1(e, w16, b2)
    return _tc_pass2(e, w16, b2, m, s)
